# Initial kernel scaffold; baseline (speedup 1.0000x reference)
import jax
import jax.numpy as jnp
from jax.experimental import pallas as pl


def kernel(x, conv1_w, conv1_b, fc2_w, fc2_b, fc3_w, fc3_b, gat1_wl, gat1_bl, gat1_wr, gat1_br, gat1_att, gat1_bias, gat2_wl, gat2_bl, gat2_wr, gat2_br, gat2_att, gat2_bias, gat3_wl, gat3_bl, gat3_wr, gat3_br, gat3_att, gat3_bias, gat4_wl, gat4_bl, gat4_wr, gat4_br, gat4_att, gat4_bias):
    raise NotImplementedError("write your pallas kernel here")


# speedup vs baseline: 29.3000x; 29.3000x over previous
import functools

import numpy as np
import jax
import jax.numpy as jnp
from jax import lax
from jax.experimental import pallas as pl
from jax.experimental.pallas import tpu as pltpu
from jax.experimental.pallas import tpu_sc as plsc

IN_CH = 256
REPR = 256
KS = 5
STRIDE = 5
HEADS = 8
K = 3
H = 384
W = 384
NH = (H - KS) // STRIDE + 1
NW = (W - KS) // STRIDE + 1
N = NH * NW
RB = 152
GRID = N // RB
PADN = 6144
NEG = -1e30


def _spatial_neighbors():
    centers = np.array(
        [[float(i), float(j)]
         for i in range(0, H - H % STRIDE, STRIDE)
         for j in range(0, W - W % STRIDE, STRIDE)], dtype=np.float32)
    diff = centers[:, None, :] - centers[None, :, :]
    dist = np.sqrt((diff * diff).sum(-1))
    idx = np.argsort(dist, axis=1, kind="stable")[:, :K]
    vals = np.take_along_axis(dist, idx, axis=1)
    rr = np.arange(N)[:, None]
    mask = (vals != 0) & (rr <= idx)
    return idx.astype(np.int32), mask.astype(np.float32)


_SP_IDX, _SP_MASK = _spatial_neighbors()
_SP_MSK8 = np.concatenate(
    [_SP_MASK, np.ones((N, 1), np.float32), np.zeros((N, 4), np.float32)], 1)
_SP_IDX3 = np.zeros((3, PADN), np.int32)
_SP_IDX3[:, :N] = _SP_IDX.T


def _embed_body(p_ref, wc_ref, b1_ref, w2_ref, b2_ref, o_ref):
    a = jnp.dot(p_ref[...], wc_ref[...], preferred_element_type=jnp.float32)
    a = jnp.maximum(a + b1_ref[...], 0.0)
    e = jnp.dot(a, w2_ref[...], preferred_element_type=jnp.float32)
    o_ref[...] = jnp.maximum(e + b2_ref[...], 0.0)


def _embed(p, wc_t, b1, w2_t, b2):
    return pl.pallas_call(
        _embed_body,
        grid=(GRID,),
        in_specs=[
            pl.BlockSpec((RB, IN_CH * KS * KS), lambda i: (i, 0)),
            pl.BlockSpec((IN_CH * KS * KS, REPR), lambda i: (0, 0)),
            pl.BlockSpec((1, REPR), lambda i: (0, 0)),
            pl.BlockSpec((REPR, REPR), lambda i: (0, 0)),
            pl.BlockSpec((1, REPR), lambda i: (0, 0)),
        ],
        out_specs=pl.BlockSpec((RB, REPR), lambda i: (i, 0)),
        out_shape=jax.ShapeDtypeStruct((N, REPR), jnp.float32),
    )(p, wc_t, b1, w2_t, b2)


def _topk_body(e_ref, et_ref, idx_ref, msk_ref):
    s = jnp.dot(e_ref[...], et_ref[...], preferred_element_type=jnp.float32)
    iota = lax.broadcasted_iota(jnp.int32, (RB, N), 1)
    rows = RB * pl.program_id(0) + lax.broadcasted_iota(jnp.int32, (RB, 1), 0)
    js, vs = [], []
    for _ in range(K):
        m = jnp.max(s, axis=1, keepdims=True)
        j = jnp.min(jnp.where(s == m, iota, jnp.int32(2 ** 30)),
                    axis=1, keepdims=True)
        s = jnp.where(iota == j, NEG, s)
        vs.append(m)
        js.append(j)
    masks = [((vs[t] != 0.0) & (rows <= js[t])).astype(jnp.float32)
             for t in range(K)]
    idx_ref[...] = jnp.concatenate(js + [jnp.zeros((RB, 5), jnp.int32)], 1)
    msk_ref[...] = jnp.concatenate(
        masks + [jnp.ones((RB, 1), jnp.float32),
                 jnp.zeros((RB, 4), jnp.float32)], 1)


def _topk(emb, emb_t):
    return pl.pallas_call(
        _topk_body,
        grid=(GRID,),
        in_specs=[
            pl.BlockSpec((RB, REPR), lambda i: (i, 0)),
            pl.BlockSpec((REPR, N), lambda i: (0, 0)),
        ],
        out_specs=[
            pl.BlockSpec((RB, 8), lambda i: (i, 0)),
            pl.BlockSpec((RB, 8), lambda i: (i, 0)),
        ],
        out_shape=[
            jax.ShapeDtypeStruct((N, 8), jnp.int32),
            jax.ShapeDtypeStruct((N, 8), jnp.float32),
        ],
    )(emb, emb_t)


def _sc_gather(table, idx, d):
    info = plsc.get_sparse_core_info()
    nc, ns = info.num_cores, info.num_subcores
    nw = nc * ns
    chunk = PADN // nw
    mesh = plsc.VectorSubcoreMesh(core_axis_name="c", subcore_axis_name="s")

    @functools.partial(
        pl.kernel,
        mesh=mesh,
        out_type=jax.ShapeDtypeStruct((3, PADN, d), jnp.float32),
        scratch_types=[
            pltpu.VMEM((chunk,), jnp.int32),
            pltpu.VMEM((chunk, d), jnp.float32),
            pltpu.SemaphoreType.DMA,
        ],
    )
    def gather_k(table_hbm, idx_hbm, out_hbm, idx_v, rows_v, sem):
        wid = lax.axis_index("s") * nc + lax.axis_index("c")
        base = wid * chunk
        for k in range(3):
            pltpu.sync_copy(idx_hbm.at[pl.ds(k * PADN + base, chunk)], idx_v)
            pltpu.async_copy(table_hbm.at[idx_v], rows_v, sem).wait()
            pltpu.sync_copy(rows_v, out_hbm.at[k, pl.ds(base, chunk)])

    return gather_k(table, idx.reshape(3 * PADN))


def _leaky(v):
    return jnp.where(v >= 0.0, v, 0.2 * v)


def _elu(v):
    return jnp.where(v > 0.0, v, jnp.exp(jnp.minimum(v, 0.0)) - 1.0)


def _gat8_body(x_ref, g0_ref, g1_ref, g2_ref, msk_ref, wl_ref, wr_ref,
               atta_ref, ex8_ref, bias_ref, o_ref):
    wl = wl_ref[...]
    xr = jnp.dot(x_ref[...], wr_ref[...], preferred_element_type=jnp.float32)
    cand = [
        jnp.dot(g0_ref[0], wl, preferred_element_type=jnp.float32),
        jnp.dot(g1_ref[0], wl, preferred_element_type=jnp.float32),
        jnp.dot(g2_ref[0], wl, preferred_element_type=jnp.float32),
        jnp.dot(x_ref[...], wl, preferred_element_type=jnp.float32),
    ]
    atta = atta_ref[...]
    msk = msk_ref[...]
    al = []
    for c in range(4):
        a = jnp.dot(_leaky(xr + cand[c]), atta,
                    preferred_element_type=jnp.float32)
        m = msk[:, c:c + 1]
        al.append(jnp.where(m > 0.0, a, NEG))
    amax = jnp.maximum(jnp.maximum(al[0], al[1]), jnp.maximum(al[2], al[3]))
    ex = [jnp.where(msk[:, c:c + 1] > 0.0, jnp.exp(al[c] - amax), 0.0)
          for c in range(4)]
    den = ex[0] + ex[1] + ex[2] + ex[3] + 1e-16
    out = jnp.zeros_like(cand[0])
    for c in range(4):
        wf = jnp.dot(ex[c] / den, ex8_ref[...],
                     preferred_element_type=jnp.float32)
        out = out + wf * cand[c]
    o_ref[...] = _elu(out + bias_ref[...])


def _gat8(x, g, msk, wl_t, wr_t, atta, ex8, bias):
    hd = HEADS * REPR
    return pl.pallas_call(
        _gat8_body,
        grid=(GRID,),
        in_specs=[
            pl.BlockSpec((RB, REPR), lambda i: (i, 0)),
            pl.BlockSpec((1, RB, REPR), lambda i: (0, i, 0)),
            pl.BlockSpec((1, RB, REPR), lambda i: (1, i, 0)),
            pl.BlockSpec((1, RB, REPR), lambda i: (2, i, 0)),
            pl.BlockSpec((RB, 8), lambda i: (i, 0)),
            pl.BlockSpec((REPR, hd), lambda i: (0, 0)),
            pl.BlockSpec((REPR, hd), lambda i: (0, 0)),
            pl.BlockSpec((hd, HEADS), lambda i: (0, 0)),
            pl.BlockSpec((HEADS, hd), lambda i: (0, 0)),
            pl.BlockSpec((1, hd), lambda i: (0, 0)),
        ],
        out_specs=pl.BlockSpec((RB, hd), lambda i: (i, 0)),
        out_shape=jax.ShapeDtypeStruct((N, hd), jnp.float32),
    )(x, g, g, g, msk, wl_t, wr_t, atta, ex8, bias)


def _proj1_body(h_ref, wl_ref, bl_ref, wr_ref, br_ref, xl_ref, xr_ref):
    h = h_ref[...]
    xl_ref[...] = jnp.dot(h, wl_ref[...],
                          preferred_element_type=jnp.float32) + bl_ref[...]
    xr_ref[...] = jnp.dot(h, wr_ref[...],
                          preferred_element_type=jnp.float32) + br_ref[...]


def _proj1(h, wl_t, bl, wr_t, br):
    ic = h.shape[1]
    return pl.pallas_call(
        _proj1_body,
        grid=(GRID,),
        in_specs=[
            pl.BlockSpec((RB, ic), lambda i: (i, 0)),
            pl.BlockSpec((ic, REPR), lambda i: (0, 0)),
            pl.BlockSpec((1, REPR), lambda i: (0, 0)),
            pl.BlockSpec((ic, REPR), lambda i: (0, 0)),
            pl.BlockSpec((1, REPR), lambda i: (0, 0)),
        ],
        out_specs=[
            pl.BlockSpec((RB, REPR), lambda i: (i, 0)),
            pl.BlockSpec((RB, REPR), lambda i: (i, 0)),
        ],
        out_shape=[
            jax.ShapeDtypeStruct((N, REPR), jnp.float32),
            jax.ShapeDtypeStruct((N, REPR), jnp.float32),
        ],
    )(h, wl_t, bl, wr_t, br)


def _gat1_body(xl_ref, xr_ref, g0_ref, g1_ref, g2_ref, msk_ref, att_ref,
               bias_ref, o_ref):
    xr = xr_ref[...]
    att = att_ref[...]
    msk = msk_ref[...]
    cand = [g0_ref[0], g1_ref[0], g2_ref[0], xl_ref[...]]
    al = []
    for c in range(4):
        a = jnp.sum(_leaky(xr + cand[c]) * att, axis=1, keepdims=True)
        m = msk[:, c:c + 1]
        al.append(jnp.where(m > 0.0, a, NEG))
    amax = jnp.maximum(jnp.maximum(al[0], al[1]), jnp.maximum(al[2], al[3]))
    ex = [jnp.where(msk[:, c:c + 1] > 0.0, jnp.exp(al[c] - amax), 0.0)
          for c in range(4)]
    den = ex[0] + ex[1] + ex[2] + ex[3] + 1e-16
    out = jnp.zeros_like(cand[0])
    for c in range(4):
        out = out + (ex[c] / den) * cand[c]
    o_ref[...] = _elu(out + bias_ref[...])


def _gat1(xl, xr, g, msk, att, bias):
    return pl.pallas_call(
        _gat1_body,
        grid=(GRID,),
        in_specs=[
            pl.BlockSpec((RB, REPR), lambda i: (i, 0)),
            pl.BlockSpec((RB, REPR), lambda i: (i, 0)),
            pl.BlockSpec((1, RB, REPR), lambda i: (0, i, 0)),
            pl.BlockSpec((1, RB, REPR), lambda i: (1, i, 0)),
            pl.BlockSpec((1, RB, REPR), lambda i: (2, i, 0)),
            pl.BlockSpec((RB, 8), lambda i: (i, 0)),
            pl.BlockSpec((1, REPR), lambda i: (0, 0)),
            pl.BlockSpec((1, REPR), lambda i: (0, 0)),
        ],
        out_specs=pl.BlockSpec((RB, REPR), lambda i: (i, 0)),
        out_shape=jax.ShapeDtypeStruct((N, REPR), jnp.float32),
    )(xl, xr, g, g, g, msk, att, bias)


def _fc3_body(h_ref, h2_ref, w1_ref, w2_ref, b_ref, o_ref):
    h2 = h2_ref[...]
    hc = (jnp.dot(h_ref[...], w1_ref[...], preferred_element_type=jnp.float32)
          + jnp.dot(h2, w2_ref[...], preferred_element_type=jnp.float32)
          + b_ref[...])
    o_ref[...] = jnp.maximum(hc, 0.0) + h2


def _fc3(h, h2, w1_t, w2_t, b):
    return pl.pallas_call(
        _fc3_body,
        grid=(GRID,),
        in_specs=[
            pl.BlockSpec((RB, REPR), lambda i: (i, 0)),
            pl.BlockSpec((RB, REPR), lambda i: (i, 0)),
            pl.BlockSpec((REPR, REPR), lambda i: (0, 0)),
            pl.BlockSpec((REPR, REPR), lambda i: (0, 0)),
            pl.BlockSpec((1, REPR), lambda i: (0, 0)),
        ],
        out_specs=pl.BlockSpec((RB, REPR), lambda i: (i, 0)),
        out_shape=jax.ShapeDtypeStruct((N, REPR), jnp.float32),
    )(h, h2, w1_t, w2_t, b)


def _block_diag_att(att):
    eye = jnp.eye(HEADS, dtype=jnp.float32)
    return (att[:, :, None] * eye[:, None, :]).reshape(HEADS * REPR, HEADS)


def kernel(x, conv1_w, conv1_b, fc2_w, fc2_b, fc3_w, fc3_b,
           gat1_wl, gat1_bl, gat1_wr, gat1_br, gat1_att, gat1_bias,
           gat2_wl, gat2_bl, gat2_wr, gat2_br, gat2_att, gat2_bias,
           gat3_wl, gat3_bl, gat3_wr, gat3_br, gat3_att, gat3_bias,
           gat4_wl, gat4_bl, gat4_wr, gat4_br, gat4_att, gat4_bias):
    f32 = jnp.float32
    p = (x[0, :, :NH * STRIDE, :NW * STRIDE]
         .reshape(IN_CH, NH, KS, NW, KS)
         .transpose(1, 3, 0, 2, 4)
         .reshape(N, IN_CH * KS * KS))
    wc_t = conv1_w.reshape(REPR, IN_CH * KS * KS).T
    emb = _embed(p, wc_t, conv1_b[None, :], fc2_w.T, fc2_b[None, :])

    nbr8, mskF = _topk(emb, emb.T)
    idxF = jnp.zeros((3, PADN), jnp.int32).at[:, :N].set(nbr8[:, :K].T)

    idxS = jnp.asarray(_SP_IDX3)
    mskS = jnp.asarray(_SP_MSK8)

    ex8 = jnp.repeat(jnp.eye(HEADS, dtype=f32), REPR, axis=1)

    gF = _sc_gather(emb, idxF, REPR)
    h = _gat8(emb, gF, mskF, gat1_wl.T, gat1_wr.T,
              _block_diag_att(gat1_att), ex8, gat1_bias[None, :])
    xl2, xr2 = _proj1(h, gat2_wl.T, gat2_bl[None, :], gat2_wr.T,
                      gat2_br[None, :])
    g2 = _sc_gather(xl2, idxF, REPR)
    hF = _gat1(xl2, xr2, g2, mskF, gat2_att, gat2_bias[None, :])

    gS = _sc_gather(emb, idxS, REPR)
    h2 = _gat8(emb, gS, mskS, gat3_wl.T, gat3_wr.T,
               _block_diag_att(gat3_att), ex8, gat3_bias[None, :])
    xl4, xr4 = _proj1(h2, gat4_wl.T, gat4_bl[None, :], gat4_wr.T,
                      gat4_br[None, :])
    g4 = _sc_gather(xl4, idxS, REPR)
    hS = _gat1(xl4, xr4, g4, mskS, gat4_att, gat4_bias[None, :])

    return _fc3(hF, hS, fc3_w[:, :REPR].T, fc3_w[:, REPR:].T, fc3_b[None, :])


# speedup vs baseline: 30.4431x; 1.0390x over previous
import functools

import numpy as np
import jax
import jax.numpy as jnp
from jax import lax
from jax.experimental import pallas as pl
from jax.experimental.pallas import tpu as pltpu
from jax.experimental.pallas import tpu_sc as plsc

IN_CH = 256
REPR = 256
KS = 5
STRIDE = 5
HEADS = 8
K = 3
H = 384
W = 384
NH = (H - KS) // STRIDE + 1
NW = (W - KS) // STRIDE + 1
N = NH * NW
RB = 152
GRID = N // RB
PADN = 6144
NEG = -1e30


def _spatial_neighbors():
    centers = np.array(
        [[float(i), float(j)]
         for i in range(0, H - H % STRIDE, STRIDE)
         for j in range(0, W - W % STRIDE, STRIDE)], dtype=np.float32)
    diff = centers[:, None, :] - centers[None, :, :]
    dist = np.sqrt((diff * diff).sum(-1))
    idx = np.argsort(dist, axis=1, kind="stable")[:, :K]
    vals = np.take_along_axis(dist, idx, axis=1)
    rr = np.arange(N)[:, None]
    mask = (vals != 0) & (rr <= idx)
    return idx.astype(np.int32), mask.astype(np.float32)


_SP_IDX, _SP_MASK = _spatial_neighbors()
_SP_MSK8 = np.concatenate(
    [_SP_MASK, np.ones((N, 1), np.float32), np.zeros((N, 4), np.float32)], 1)
_SP_SEL = np.zeros((3, N, RB), np.float32)
for _c in range(K):
    for _i in range(N):
        if _SP_MASK[_i, _c] > 0:
            _loc = _SP_IDX[_i, _c] - RB * (_i // RB)
            assert 0 <= _loc < RB
            _SP_SEL[_c, _i, _loc] = 1.0


def _embed_body(p_ref, wc_ref, b1_ref, w2_ref, b2_ref, o_ref):
    a = jnp.dot(p_ref[...], wc_ref[...], preferred_element_type=jnp.float32)
    a = jnp.maximum(a + b1_ref[...], 0.0)
    e = jnp.dot(a, w2_ref[...], preferred_element_type=jnp.float32)
    o_ref[...] = jnp.maximum(e + b2_ref[...], 0.0)


def _embed(p, wc_t, b1, w2_t, b2):
    return pl.pallas_call(
        _embed_body,
        grid=(GRID,),
        in_specs=[
            pl.BlockSpec((RB, IN_CH * KS * KS), lambda i: (i, 0)),
            pl.BlockSpec((IN_CH * KS * KS, REPR), lambda i: (0, 0)),
            pl.BlockSpec((1, REPR), lambda i: (0, 0)),
            pl.BlockSpec((REPR, REPR), lambda i: (0, 0)),
            pl.BlockSpec((1, REPR), lambda i: (0, 0)),
        ],
        out_specs=pl.BlockSpec((RB, REPR), lambda i: (i, 0)),
        out_shape=jax.ShapeDtypeStruct((N, REPR), jnp.float32),
    )(p, wc_t, b1, w2_t, b2)


def _topk_body(e_ref, et_ref, idx_ref, msk_ref):
    s = lax.dot_general(e_ref[...], et_ref[...], (((1,), (1,)), ((), ())),
                        preferred_element_type=jnp.float32)
    iota = lax.broadcasted_iota(jnp.int32, (RB, N), 1)
    rows = RB * pl.program_id(0) + lax.broadcasted_iota(jnp.int32, (RB, 1), 0)
    js, vs = [], []
    for _ in range(K):
        m = jnp.max(s, axis=1, keepdims=True)
        j = jnp.min(jnp.where(s == m, iota, jnp.int32(2 ** 30)),
                    axis=1, keepdims=True)
        s = jnp.where(iota == j, NEG, s)
        vs.append(m)
        js.append(j)
    masks = [((vs[t] != 0.0) & (rows <= js[t])).astype(jnp.float32)
             for t in range(K)]
    idx_ref[...] = jnp.concatenate(js + [jnp.zeros((RB, 5), jnp.int32)], 1)
    msk_ref[...] = jnp.concatenate(
        masks + [jnp.ones((RB, 1), jnp.float32),
                 jnp.zeros((RB, 4), jnp.float32)], 1)


def _topk(emb):
    return pl.pallas_call(
        _topk_body,
        grid=(GRID,),
        in_specs=[
            pl.BlockSpec((RB, REPR), lambda i: (i, 0)),
            pl.BlockSpec((N, REPR), lambda i: (0, 0)),
        ],
        out_specs=[
            pl.BlockSpec((RB, 8), lambda i: (i, 0)),
            pl.BlockSpec((RB, 8), lambda i: (i, 0)),
        ],
        out_shape=[
            jax.ShapeDtypeStruct((N, 8), jnp.int32),
            jax.ShapeDtypeStruct((N, 8), jnp.float32),
        ],
    )(emb, emb)


def _sc_gather(table, idx, d):
    info = plsc.get_sparse_core_info()
    nc, ns = info.num_cores, info.num_subcores
    nw = nc * ns
    chunk = PADN // nw
    mesh = plsc.VectorSubcoreMesh(core_axis_name="c", subcore_axis_name="s")

    @functools.partial(
        pl.kernel,
        mesh=mesh,
        out_type=jax.ShapeDtypeStruct((3, PADN, d), jnp.float32),
        scratch_types=[
            pltpu.VMEM((chunk,), jnp.int32),
            pltpu.VMEM((chunk, d), jnp.float32),
            pltpu.VMEM((chunk, d), jnp.float32),
            pltpu.SemaphoreType.DMA,
            pltpu.SemaphoreType.DMA,
            pltpu.SemaphoreType.DMA,
        ],
    )
    def gather_k(table_hbm, idx_hbm, out_hbm, idx_v, rows_a, rows_b,
                 gsem, wsem_a, wsem_b):
        wid = lax.axis_index("s") * nc + lax.axis_index("c")
        base = wid * chunk
        bufs = (rows_a, rows_b)
        wsems = (wsem_a, wsem_b)
        wb = [None, None]
        for k in range(3):
            buf, wsem = bufs[k % 2], wsems[k % 2]
            if wb[k % 2] is not None:
                wb[k % 2].wait()
            pltpu.sync_copy(idx_hbm.at[pl.ds(k * PADN + base, chunk)], idx_v)
            pltpu.async_copy(table_hbm.at[idx_v], buf, gsem).wait()
            wb[k % 2] = pltpu.async_copy(
                buf, out_hbm.at[k, pl.ds(base, chunk)], wsem)
        wb[1].wait()
        wb[0].wait()

    return gather_k(table, idx.reshape(3 * PADN))


def _leaky(v):
    return jnp.where(v >= 0.0, v, 0.2 * v)


def _elu(v):
    return jnp.where(v > 0.0, v, jnp.exp(jnp.minimum(v, 0.0)) - 1.0)


def _gat8_body(x_ref, g0_ref, g1_ref, g2_ref, msk_ref, wl_ref, wr_ref,
               atta_ref, ex8_ref, bias_ref, o_ref):
    wl = wl_ref[...]
    xr = jnp.dot(x_ref[...], wr_ref[...], preferred_element_type=jnp.float32)
    cand = [
        jnp.dot(g0_ref[0], wl, preferred_element_type=jnp.float32),
        jnp.dot(g1_ref[0], wl, preferred_element_type=jnp.float32),
        jnp.dot(g2_ref[0], wl, preferred_element_type=jnp.float32),
        jnp.dot(x_ref[...], wl, preferred_element_type=jnp.float32),
    ]
    atta = atta_ref[...]
    msk = msk_ref[...]
    al = []
    for c in range(4):
        a = jnp.dot(_leaky(xr + cand[c]), atta,
                    preferred_element_type=jnp.float32)
        m = msk[:, c:c + 1]
        al.append(jnp.where(m > 0.0, a, NEG))
    amax = jnp.maximum(jnp.maximum(al[0], al[1]), jnp.maximum(al[2], al[3]))
    ex = [jnp.where(msk[:, c:c + 1] > 0.0, jnp.exp(al[c] - amax), 0.0)
          for c in range(4)]
    den = ex[0] + ex[1] + ex[2] + ex[3] + 1e-16
    out = jnp.zeros_like(cand[0])
    for c in range(4):
        wf = jnp.dot(ex[c] / den, ex8_ref[...],
                     preferred_element_type=jnp.float32)
        out = out + wf * cand[c]
    o_ref[...] = _elu(out + bias_ref[...])


def _gat8(x, g, msk, wl_t, wr_t, atta, ex8, bias):
    hd = HEADS * REPR
    return pl.pallas_call(
        _gat8_body,
        grid=(GRID,),
        in_specs=[
            pl.BlockSpec((RB, REPR), lambda i: (i, 0)),
            pl.BlockSpec((1, RB, REPR), lambda i: (0, i, 0)),
            pl.BlockSpec((1, RB, REPR), lambda i: (1, i, 0)),
            pl.BlockSpec((1, RB, REPR), lambda i: (2, i, 0)),
            pl.BlockSpec((RB, 8), lambda i: (i, 0)),
            pl.BlockSpec((REPR, hd), lambda i: (0, 0)),
            pl.BlockSpec((REPR, hd), lambda i: (0, 0)),
            pl.BlockSpec((hd, HEADS), lambda i: (0, 0)),
            pl.BlockSpec((HEADS, hd), lambda i: (0, 0)),
            pl.BlockSpec((1, hd), lambda i: (0, 0)),
        ],
        out_specs=pl.BlockSpec((RB, hd), lambda i: (i, 0)),
        out_shape=jax.ShapeDtypeStruct((N, hd), jnp.float32),
    )(x, g, g, g, msk, wl_t, wr_t, atta, ex8, bias)


def _proj1_body(h_ref, wl_ref, bl_ref, wr_ref, br_ref, xl_ref, xr_ref):
    h = h_ref[...]
    xl_ref[...] = jnp.dot(h, wl_ref[...],
                          preferred_element_type=jnp.float32) + bl_ref[...]
    xr_ref[...] = jnp.dot(h, wr_ref[...],
                          preferred_element_type=jnp.float32) + br_ref[...]


def _proj1(h, wl_t, bl, wr_t, br):
    ic = h.shape[1]
    return pl.pallas_call(
        _proj1_body,
        grid=(GRID,),
        in_specs=[
            pl.BlockSpec((RB, ic), lambda i: (i, 0)),
            pl.BlockSpec((ic, REPR), lambda i: (0, 0)),
            pl.BlockSpec((1, REPR), lambda i: (0, 0)),
            pl.BlockSpec((ic, REPR), lambda i: (0, 0)),
            pl.BlockSpec((1, REPR), lambda i: (0, 0)),
        ],
        out_specs=[
            pl.BlockSpec((RB, REPR), lambda i: (i, 0)),
            pl.BlockSpec((RB, REPR), lambda i: (i, 0)),
        ],
        out_shape=[
            jax.ShapeDtypeStruct((N, REPR), jnp.float32),
            jax.ShapeDtypeStruct((N, REPR), jnp.float32),
        ],
    )(h, wl_t, bl, wr_t, br)


def _gat1_body(xl_ref, xr_ref, g0_ref, g1_ref, g2_ref, msk_ref, att_ref,
               bias_ref, o_ref):
    xr = xr_ref[...]
    att = att_ref[...]
    msk = msk_ref[...]
    cand = [g0_ref[0], g1_ref[0], g2_ref[0], xl_ref[...]]
    al = []
    for c in range(4):
        a = jnp.sum(_leaky(xr + cand[c]) * att, axis=1, keepdims=True)
        m = msk[:, c:c + 1]
        al.append(jnp.where(m > 0.0, a, NEG))
    amax = jnp.maximum(jnp.maximum(al[0], al[1]), jnp.maximum(al[2], al[3]))
    ex = [jnp.where(msk[:, c:c + 1] > 0.0, jnp.exp(al[c] - amax), 0.0)
          for c in range(4)]
    den = ex[0] + ex[1] + ex[2] + ex[3] + 1e-16
    out = jnp.zeros_like(cand[0])
    for c in range(4):
        out = out + (ex[c] / den) * cand[c]
    o_ref[...] = _elu(out + bias_ref[...])


def _gat1(xl, xr, g, msk, att, bias):
    return pl.pallas_call(
        _gat1_body,
        grid=(GRID,),
        in_specs=[
            pl.BlockSpec((RB, REPR), lambda i: (i, 0)),
            pl.BlockSpec((RB, REPR), lambda i: (i, 0)),
            pl.BlockSpec((1, RB, REPR), lambda i: (0, i, 0)),
            pl.BlockSpec((1, RB, REPR), lambda i: (1, i, 0)),
            pl.BlockSpec((1, RB, REPR), lambda i: (2, i, 0)),
            pl.BlockSpec((RB, 8), lambda i: (i, 0)),
            pl.BlockSpec((1, REPR), lambda i: (0, 0)),
            pl.BlockSpec((1, REPR), lambda i: (0, 0)),
        ],
        out_specs=pl.BlockSpec((RB, REPR), lambda i: (i, 0)),
        out_shape=jax.ShapeDtypeStruct((N, REPR), jnp.float32),
    )(xl, xr, g, g, g, msk, att, bias)


def _gat8_sp_body(x_ref, msk_ref, s_ref, wl_ref, wr_ref, atta_ref, ex8_ref,
                  bias_ref, o_ref):
    xl = jnp.dot(x_ref[...], wl_ref[...], preferred_element_type=jnp.float32)
    xr = jnp.dot(x_ref[...], wr_ref[...], preferred_element_type=jnp.float32)
    msk = msk_ref[...]
    cand = [jnp.dot(s_ref[c], xl, preferred_element_type=jnp.float32)
            for c in range(3)] + [xl]
    al = []
    for c in range(4):
        a = jnp.dot(_leaky(xr + cand[c]), atta_ref[...],
                    preferred_element_type=jnp.float32)
        al.append(jnp.where(msk[:, c:c + 1] > 0.0, a, NEG))
    amax = jnp.maximum(jnp.maximum(al[0], al[1]), jnp.maximum(al[2], al[3]))
    ex = [jnp.where(msk[:, c:c + 1] > 0.0, jnp.exp(al[c] - amax), 0.0)
          for c in range(4)]
    den = ex[0] + ex[1] + ex[2] + ex[3] + 1e-16
    out = jnp.zeros_like(xl)
    for c in range(4):
        wf = jnp.dot(ex[c] / den, ex8_ref[...],
                     preferred_element_type=jnp.float32)
        out = out + wf * cand[c]
    o_ref[...] = _elu(out + bias_ref[...])


def _gat8_sp(x, msk, sel, wl_t, wr_t, atta, ex8, bias):
    hd = HEADS * REPR
    return pl.pallas_call(
        _gat8_sp_body,
        grid=(GRID,),
        in_specs=[
            pl.BlockSpec((RB, REPR), lambda i: (i, 0)),
            pl.BlockSpec((RB, 8), lambda i: (i, 0)),
            pl.BlockSpec((3, RB, RB), lambda i: (0, i, 0)),
            pl.BlockSpec((REPR, hd), lambda i: (0, 0)),
            pl.BlockSpec((REPR, hd), lambda i: (0, 0)),
            pl.BlockSpec((hd, HEADS), lambda i: (0, 0)),
            pl.BlockSpec((HEADS, hd), lambda i: (0, 0)),
            pl.BlockSpec((1, hd), lambda i: (0, 0)),
        ],
        out_specs=pl.BlockSpec((RB, hd), lambda i: (i, 0)),
        out_shape=jax.ShapeDtypeStruct((N, hd), jnp.float32),
    )(x, msk, sel, wl_t, wr_t, atta, ex8, bias)


def _gat1_sp_body(h_ref, msk_ref, s_ref, wl_ref, bl_ref, wr_ref, br_ref,
                  att_ref, bias_ref, o_ref):
    xl = jnp.dot(h_ref[...], wl_ref[...],
                 preferred_element_type=jnp.float32) + bl_ref[...]
    xr = jnp.dot(h_ref[...], wr_ref[...],
                 preferred_element_type=jnp.float32) + br_ref[...]
    msk = msk_ref[...]
    att = att_ref[...]
    cand = [jnp.dot(s_ref[c], xl, preferred_element_type=jnp.float32)
            for c in range(3)] + [xl]
    al = []
    for c in range(4):
        a = jnp.sum(_leaky(xr + cand[c]) * att, axis=1, keepdims=True)
        al.append(jnp.where(msk[:, c:c + 1] > 0.0, a, NEG))
    amax = jnp.maximum(jnp.maximum(al[0], al[1]), jnp.maximum(al[2], al[3]))
    ex = [jnp.where(msk[:, c:c + 1] > 0.0, jnp.exp(al[c] - amax), 0.0)
          for c in range(4)]
    den = ex[0] + ex[1] + ex[2] + ex[3] + 1e-16
    out = jnp.zeros_like(xl)
    for c in range(4):
        out = out + (ex[c] / den) * cand[c]
    o_ref[...] = _elu(out + bias_ref[...])


def _gat1_sp(h, msk, sel, wl_t, bl, wr_t, br, att, bias):
    ic = h.shape[1]
    return pl.pallas_call(
        _gat1_sp_body,
        grid=(GRID,),
        in_specs=[
            pl.BlockSpec((RB, ic), lambda i: (i, 0)),
            pl.BlockSpec((RB, 8), lambda i: (i, 0)),
            pl.BlockSpec((3, RB, RB), lambda i: (0, i, 0)),
            pl.BlockSpec((ic, REPR), lambda i: (0, 0)),
            pl.BlockSpec((1, REPR), lambda i: (0, 0)),
            pl.BlockSpec((ic, REPR), lambda i: (0, 0)),
            pl.BlockSpec((1, REPR), lambda i: (0, 0)),
            pl.BlockSpec((1, REPR), lambda i: (0, 0)),
            pl.BlockSpec((1, REPR), lambda i: (0, 0)),
        ],
        out_specs=pl.BlockSpec((RB, REPR), lambda i: (i, 0)),
        out_shape=jax.ShapeDtypeStruct((N, REPR), jnp.float32),
    )(h, msk, sel, wl_t, bl, wr_t, br, att, bias)


def _fc3_body(h_ref, h2_ref, w1_ref, w2_ref, b_ref, o_ref):
    h2 = h2_ref[...]
    hc = (jnp.dot(h_ref[...], w1_ref[...], preferred_element_type=jnp.float32)
          + jnp.dot(h2, w2_ref[...], preferred_element_type=jnp.float32)
          + b_ref[...])
    o_ref[...] = jnp.maximum(hc, 0.0) + h2


def _fc3(h, h2, w1_t, w2_t, b):
    return pl.pallas_call(
        _fc3_body,
        grid=(GRID,),
        in_specs=[
            pl.BlockSpec((RB, REPR), lambda i: (i, 0)),
            pl.BlockSpec((RB, REPR), lambda i: (i, 0)),
            pl.BlockSpec((REPR, REPR), lambda i: (0, 0)),
            pl.BlockSpec((REPR, REPR), lambda i: (0, 0)),
            pl.BlockSpec((1, REPR), lambda i: (0, 0)),
        ],
        out_specs=pl.BlockSpec((RB, REPR), lambda i: (i, 0)),
        out_shape=jax.ShapeDtypeStruct((N, REPR), jnp.float32),
    )(h, h2, w1_t, w2_t, b)


def _block_diag_att(att):
    eye = jnp.eye(HEADS, dtype=jnp.float32)
    return (att[:, :, None] * eye[:, None, :]).reshape(HEADS * REPR, HEADS)


def kernel(x, conv1_w, conv1_b, fc2_w, fc2_b, fc3_w, fc3_b,
           gat1_wl, gat1_bl, gat1_wr, gat1_br, gat1_att, gat1_bias,
           gat2_wl, gat2_bl, gat2_wr, gat2_br, gat2_att, gat2_bias,
           gat3_wl, gat3_bl, gat3_wr, gat3_br, gat3_att, gat3_bias,
           gat4_wl, gat4_bl, gat4_wr, gat4_br, gat4_att, gat4_bias):
    f32 = jnp.float32
    p = (x[0, :, :NH * STRIDE, :NW * STRIDE]
         .reshape(IN_CH, NH, KS, NW, KS)
         .transpose(1, 3, 0, 2, 4)
         .reshape(N, IN_CH * KS * KS))
    wc_t = conv1_w.reshape(REPR, IN_CH * KS * KS).T
    emb = _embed(p, wc_t, conv1_b[None, :], fc2_w.T, fc2_b[None, :])

    nbr8, mskF = _topk(emb)
    idxF = jnp.zeros((3, PADN), jnp.int32).at[:, :N].set(nbr8[:, :K].T)

    mskS = jnp.asarray(_SP_MSK8)
    selS = jnp.asarray(_SP_SEL)

    ex8 = jnp.repeat(jnp.eye(HEADS, dtype=f32), REPR, axis=1)

    gF = _sc_gather(emb, idxF, REPR)
    h = _gat8(emb, gF, mskF, gat1_wl.T, gat1_wr.T,
              _block_diag_att(gat1_att), ex8, gat1_bias[None, :])
    xl2, xr2 = _proj1(h, gat2_wl.T, gat2_bl[None, :], gat2_wr.T,
                      gat2_br[None, :])
    g2 = _sc_gather(xl2, idxF, REPR)
    hF = _gat1(xl2, xr2, g2, mskF, gat2_att, gat2_bias[None, :])

    h2 = _gat8_sp(emb, mskS, selS, gat3_wl.T, gat3_wr.T,
                  _block_diag_att(gat3_att), ex8, gat3_bias[None, :])
    hS = _gat1_sp(h2, mskS, selS, gat4_wl.T, gat4_bl[None, :], gat4_wr.T,
                  gat4_br[None, :], gat4_att, gat4_bias[None, :])

    return _fc3(hF, hS, fc3_w[:, :REPR].T, fc3_w[:, REPR:].T, fc3_b[None, :])


# speedup vs baseline: 31.7589x; 1.0432x over previous
import functools

import numpy as np
import jax
import jax.numpy as jnp
from jax import lax
from jax.experimental import pallas as pl
from jax.experimental.pallas import tpu as pltpu
from jax.experimental.pallas import tpu_sc as plsc

IN_CH = 256
REPR = 256
KS = 5
STRIDE = 5
HEADS = 8
K = 3
H = 384
W = 384
NH = (H - KS) // STRIDE + 1
NW = (W - KS) // STRIDE + 1
N = NH * NW
RB = 152
GRID = N // RB
PADN = 6144
NEG = -1e30


def _spatial_neighbors():
    centers = np.array(
        [[float(i), float(j)]
         for i in range(0, H - H % STRIDE, STRIDE)
         for j in range(0, W - W % STRIDE, STRIDE)], dtype=np.float32)
    diff = centers[:, None, :] - centers[None, :, :]
    dist = np.sqrt((diff * diff).sum(-1))
    idx = np.argsort(dist, axis=1, kind="stable")[:, :K]
    vals = np.take_along_axis(dist, idx, axis=1)
    rr = np.arange(N)[:, None]
    mask = (vals != 0) & (rr <= idx)
    return idx.astype(np.int32), mask.astype(np.float32)


_SP_IDX, _SP_MASK = _spatial_neighbors()
_SP_MSK8 = np.concatenate(
    [_SP_MASK, np.ones((N, 1), np.float32), np.zeros((N, 4), np.float32)], 1)
_SP_SEL = np.zeros((3, N, RB), np.float32)
for _c in range(K):
    for _i in range(N):
        if _SP_MASK[_i, _c] > 0:
            _loc = _SP_IDX[_i, _c] - RB * (_i // RB)
            assert 0 <= _loc < RB
            _SP_SEL[_c, _i, _loc] = 1.0


def _embed_body(p_ref, wc_ref, b1_ref, w2_ref, b2_ref, o_ref):
    a = jnp.dot(p_ref[...], wc_ref[...], preferred_element_type=jnp.float32)
    a = jnp.maximum(a + b1_ref[...], 0.0)
    e = jnp.dot(a, w2_ref[...], preferred_element_type=jnp.float32)
    o_ref[...] = jnp.maximum(e + b2_ref[...], 0.0)


def _embed(p, wc_t, b1, w2_t, b2):
    return pl.pallas_call(
        _embed_body,
        grid=(GRID,),
        in_specs=[
            pl.BlockSpec((RB, IN_CH * KS * KS), lambda i: (i, 0)),
            pl.BlockSpec((IN_CH * KS * KS, REPR), lambda i: (0, 0)),
            pl.BlockSpec((1, REPR), lambda i: (0, 0)),
            pl.BlockSpec((REPR, REPR), lambda i: (0, 0)),
            pl.BlockSpec((1, REPR), lambda i: (0, 0)),
        ],
        out_specs=pl.BlockSpec((RB, REPR), lambda i: (i, 0)),
        out_shape=jax.ShapeDtypeStruct((N, REPR), jnp.float32),
    )(p, wc_t, b1, w2_t, b2)


def _topk_body(e_ref, et_ref, idx_ref, msk_ref):
    s = lax.dot_general(e_ref[...], et_ref[...], (((1,), (1,)), ((), ())),
                        preferred_element_type=jnp.float32)
    iota = lax.broadcasted_iota(jnp.int32, (RB, N), 1)
    rows = RB * pl.program_id(0) + lax.broadcasted_iota(jnp.int32, (RB, 1), 0)
    js, vs = [], []
    for _ in range(K):
        m = jnp.max(s, axis=1, keepdims=True)
        j = jnp.min(jnp.where(s == m, iota, jnp.int32(2 ** 30)),
                    axis=1, keepdims=True)
        s = jnp.where(iota == j, NEG, s)
        vs.append(m)
        js.append(j)
    masks = [((vs[t] != 0.0) & (rows <= js[t])).astype(jnp.float32)
             for t in range(K)]
    idx_ref[...] = jnp.concatenate(js + [jnp.zeros((RB, 5), jnp.int32)], 1)
    msk_ref[...] = jnp.concatenate(
        masks + [jnp.ones((RB, 1), jnp.float32),
                 jnp.zeros((RB, 4), jnp.float32)], 1)


def _topk(emb):
    return pl.pallas_call(
        _topk_body,
        grid=(GRID,),
        in_specs=[
            pl.BlockSpec((RB, REPR), lambda i: (i, 0)),
            pl.BlockSpec((N, REPR), lambda i: (0, 0)),
        ],
        out_specs=[
            pl.BlockSpec((RB, 8), lambda i: (i, 0)),
            pl.BlockSpec((RB, 8), lambda i: (i, 0)),
        ],
        out_shape=[
            jax.ShapeDtypeStruct((N, 8), jnp.int32),
            jax.ShapeDtypeStruct((N, 8), jnp.float32),
        ],
    )(emb, emb)


def _sc_gather(table, idx, d):
    info = plsc.get_sparse_core_info()
    nc, ns = info.num_cores, info.num_subcores
    nw = nc * ns
    chunk = PADN // nw
    mesh = plsc.VectorSubcoreMesh(core_axis_name="c", subcore_axis_name="s")

    @functools.partial(
        pl.kernel,
        mesh=mesh,
        out_type=jax.ShapeDtypeStruct((3, PADN, d), jnp.float32),
        scratch_types=[
            pltpu.VMEM((chunk,), jnp.int32),
            pltpu.VMEM((chunk, d), jnp.float32),
            pltpu.VMEM((chunk, d), jnp.float32),
            pltpu.SemaphoreType.DMA,
            pltpu.SemaphoreType.DMA,
            pltpu.SemaphoreType.DMA,
        ],
    )
    def gather_k(table_hbm, idx_hbm, out_hbm, idx_v, rows_a, rows_b,
                 gsem, wsem_a, wsem_b):
        wid = lax.axis_index("s") * nc + lax.axis_index("c")
        base = wid * chunk
        bufs = (rows_a, rows_b)
        wsems = (wsem_a, wsem_b)
        wb = [None, None]
        for k in range(3):
            buf, wsem = bufs[k % 2], wsems[k % 2]
            if wb[k % 2] is not None:
                wb[k % 2].wait()
            pltpu.sync_copy(idx_hbm.at[pl.ds(k * PADN + base, chunk)], idx_v)
            pltpu.async_copy(table_hbm.at[idx_v], buf, gsem).wait()
            wb[k % 2] = pltpu.async_copy(
                buf, out_hbm.at[k, pl.ds(base, chunk)], wsem)
        wb[1].wait()
        wb[0].wait()

    return gather_k(table, idx.reshape(3 * PADN))


def _leaky(v):
    return jnp.where(v >= 0.0, v, 0.2 * v)


def _elu(v):
    return jnp.where(v > 0.0, v, jnp.exp(jnp.minimum(v, 0.0)) - 1.0)


def _gat8_attention(x, g0, g1, g2, msk, wl, wr, atta, ex8, bias):
    xr = jnp.dot(x, wr, preferred_element_type=jnp.float32)
    cand = [
        jnp.dot(g0, wl, preferred_element_type=jnp.float32),
        jnp.dot(g1, wl, preferred_element_type=jnp.float32),
        jnp.dot(g2, wl, preferred_element_type=jnp.float32),
        jnp.dot(x, wl, preferred_element_type=jnp.float32),
    ]
    al = []
    for c in range(4):
        a = jnp.dot(_leaky(xr + cand[c]), atta,
                    preferred_element_type=jnp.float32)
        al.append(jnp.where(msk[:, c:c + 1] > 0.0, a, NEG))
    amax = jnp.maximum(jnp.maximum(al[0], al[1]), jnp.maximum(al[2], al[3]))
    ex = [jnp.where(msk[:, c:c + 1] > 0.0, jnp.exp(al[c] - amax), 0.0)
          for c in range(4)]
    den = ex[0] + ex[1] + ex[2] + ex[3] + 1e-16
    out = jnp.zeros_like(cand[0])
    for c in range(4):
        wf = jnp.dot(ex[c] / den, ex8,
                     preferred_element_type=jnp.float32)
        out = out + wf * cand[c]
    return _elu(out + bias)


def _gat8f_body(x_ref, g0_ref, g1_ref, g2_ref, msk_ref, wl_ref, wr_ref,
                atta_ref, ex8_ref, bias_ref, wl2_ref, bl2_ref, wr2_ref,
                br2_ref, xl2_ref, xr2_ref):
    h = _gat8_attention(x_ref[...], g0_ref[0], g1_ref[0], g2_ref[0],
                        msk_ref[...], wl_ref[...], wr_ref[...], atta_ref[...],
                        ex8_ref[...], bias_ref[...])
    xl2_ref[...] = jnp.dot(h, wl2_ref[...],
                           preferred_element_type=jnp.float32) + bl2_ref[...]
    xr2_ref[...] = jnp.dot(h, wr2_ref[...],
                           preferred_element_type=jnp.float32) + br2_ref[...]


def _gat8f(x, g, msk, wl_t, wr_t, atta, ex8, bias, wl2_t, bl2, wr2_t, br2):
    hd = HEADS * REPR
    return pl.pallas_call(
        _gat8f_body,
        grid=(GRID,),
        in_specs=[
            pl.BlockSpec((RB, REPR), lambda i: (i, 0)),
            pl.BlockSpec((1, RB, REPR), lambda i: (0, i, 0)),
            pl.BlockSpec((1, RB, REPR), lambda i: (1, i, 0)),
            pl.BlockSpec((1, RB, REPR), lambda i: (2, i, 0)),
            pl.BlockSpec((RB, 8), lambda i: (i, 0)),
            pl.BlockSpec((REPR, hd), lambda i: (0, 0)),
            pl.BlockSpec((REPR, hd), lambda i: (0, 0)),
            pl.BlockSpec((hd, HEADS), lambda i: (0, 0)),
            pl.BlockSpec((HEADS, hd), lambda i: (0, 0)),
            pl.BlockSpec((1, hd), lambda i: (0, 0)),
            pl.BlockSpec((hd, REPR), lambda i: (0, 0)),
            pl.BlockSpec((1, REPR), lambda i: (0, 0)),
            pl.BlockSpec((hd, REPR), lambda i: (0, 0)),
            pl.BlockSpec((1, REPR), lambda i: (0, 0)),
        ],
        out_specs=[
            pl.BlockSpec((RB, REPR), lambda i: (i, 0)),
            pl.BlockSpec((RB, REPR), lambda i: (i, 0)),
        ],
        out_shape=[
            jax.ShapeDtypeStruct((N, REPR), jnp.float32),
            jax.ShapeDtypeStruct((N, REPR), jnp.float32),
        ],
    )(x, g, g, g, msk, wl_t, wr_t, atta, ex8, bias, wl2_t, bl2, wr2_t, br2)


def _gat1_attention(xl, xr, g0, g1, g2, msk, att, bias):
    cand = [g0, g1, g2, xl]
    al = []
    for c in range(4):
        a = jnp.sum(_leaky(xr + cand[c]) * att, axis=1, keepdims=True)
        al.append(jnp.where(msk[:, c:c + 1] > 0.0, a, NEG))
    amax = jnp.maximum(jnp.maximum(al[0], al[1]), jnp.maximum(al[2], al[3]))
    ex = [jnp.where(msk[:, c:c + 1] > 0.0, jnp.exp(al[c] - amax), 0.0)
          for c in range(4)]
    den = ex[0] + ex[1] + ex[2] + ex[3] + 1e-16
    out = jnp.zeros_like(xl)
    for c in range(4):
        out = out + (ex[c] / den) * cand[c]
    return _elu(out + bias)


def _gat1f_body(xl_ref, xr_ref, g0_ref, g1_ref, g2_ref, msk_ref, att_ref,
                bias_ref, hs_ref, w1_ref, w2_ref, fb_ref, o_ref):
    hf = _gat1_attention(xl_ref[...], xr_ref[...], g0_ref[0], g1_ref[0],
                         g2_ref[0], msk_ref[...], att_ref[...], bias_ref[...])
    hs = hs_ref[...]
    hc = (jnp.dot(hf, w1_ref[...], preferred_element_type=jnp.float32)
          + jnp.dot(hs, w2_ref[...], preferred_element_type=jnp.float32)
          + fb_ref[...])
    o_ref[...] = jnp.maximum(hc, 0.0) + hs


def _gat1f_fc3(xl, xr, g, msk, att, bias, hs, w1_t, w2_t, fb):
    return pl.pallas_call(
        _gat1f_body,
        grid=(GRID,),
        in_specs=[
            pl.BlockSpec((RB, REPR), lambda i: (i, 0)),
            pl.BlockSpec((RB, REPR), lambda i: (i, 0)),
            pl.BlockSpec((1, RB, REPR), lambda i: (0, i, 0)),
            pl.BlockSpec((1, RB, REPR), lambda i: (1, i, 0)),
            pl.BlockSpec((1, RB, REPR), lambda i: (2, i, 0)),
            pl.BlockSpec((RB, 8), lambda i: (i, 0)),
            pl.BlockSpec((1, REPR), lambda i: (0, 0)),
            pl.BlockSpec((1, REPR), lambda i: (0, 0)),
            pl.BlockSpec((RB, REPR), lambda i: (i, 0)),
            pl.BlockSpec((REPR, REPR), lambda i: (0, 0)),
            pl.BlockSpec((REPR, REPR), lambda i: (0, 0)),
            pl.BlockSpec((1, REPR), lambda i: (0, 0)),
        ],
        out_specs=pl.BlockSpec((RB, REPR), lambda i: (i, 0)),
        out_shape=jax.ShapeDtypeStruct((N, REPR), jnp.float32),
    )(xl, xr, g, g, g, msk, att, bias, hs, w1_t, w2_t, fb)


def _spatial_body(x_ref, msk_ref, s_ref, wl3_ref, wr3_ref, atta_ref, ex8_ref,
                  b3_ref, wl4_ref, bl4_ref, wr4_ref, br4_ref, att4_ref,
                  b4_ref, o_ref):
    x = x_ref[...]
    msk = msk_ref[...]
    sx = [jnp.dot(s_ref[c], x, preferred_element_type=jnp.float32)
          for c in range(3)]
    h2 = _gat8_attention(x, sx[0], sx[1], sx[2], msk, wl3_ref[...],
                         wr3_ref[...], atta_ref[...], ex8_ref[...], b3_ref[...])
    xl4 = jnp.dot(h2, wl4_ref[...],
                  preferred_element_type=jnp.float32) + bl4_ref[...]
    xr4 = jnp.dot(h2, wr4_ref[...],
                  preferred_element_type=jnp.float32) + br4_ref[...]
    g4 = [jnp.dot(s_ref[c], xl4, preferred_element_type=jnp.float32)
          for c in range(3)]
    o_ref[...] = _gat1_attention(xl4, xr4, g4[0], g4[1], g4[2], msk,
                                 att4_ref[...], b4_ref[...])


def _spatial(x, msk, sel, wl3_t, wr3_t, atta, ex8, b3, wl4_t, bl4, wr4_t,
             br4, att4, b4):
    hd = HEADS * REPR
    return pl.pallas_call(
        _spatial_body,
        grid=(GRID,),
        in_specs=[
            pl.BlockSpec((RB, REPR), lambda i: (i, 0)),
            pl.BlockSpec((RB, 8), lambda i: (i, 0)),
            pl.BlockSpec((3, RB, RB), lambda i: (0, i, 0)),
            pl.BlockSpec((REPR, hd), lambda i: (0, 0)),
            pl.BlockSpec((REPR, hd), lambda i: (0, 0)),
            pl.BlockSpec((hd, HEADS), lambda i: (0, 0)),
            pl.BlockSpec((HEADS, hd), lambda i: (0, 0)),
            pl.BlockSpec((1, hd), lambda i: (0, 0)),
            pl.BlockSpec((hd, REPR), lambda i: (0, 0)),
            pl.BlockSpec((1, REPR), lambda i: (0, 0)),
            pl.BlockSpec((hd, REPR), lambda i: (0, 0)),
            pl.BlockSpec((1, REPR), lambda i: (0, 0)),
            pl.BlockSpec((1, REPR), lambda i: (0, 0)),
            pl.BlockSpec((1, REPR), lambda i: (0, 0)),
        ],
        out_specs=pl.BlockSpec((RB, REPR), lambda i: (i, 0)),
        out_shape=jax.ShapeDtypeStruct((N, REPR), jnp.float32),
    )(x, msk, sel, wl3_t, wr3_t, atta, ex8, b3, wl4_t, bl4, wr4_t, br4,
      att4, b4)


def _block_diag_att(att):
    eye = jnp.eye(HEADS, dtype=jnp.float32)
    return (att[:, :, None] * eye[:, None, :]).reshape(HEADS * REPR, HEADS)


def kernel(x, conv1_w, conv1_b, fc2_w, fc2_b, fc3_w, fc3_b,
           gat1_wl, gat1_bl, gat1_wr, gat1_br, gat1_att, gat1_bias,
           gat2_wl, gat2_bl, gat2_wr, gat2_br, gat2_att, gat2_bias,
           gat3_wl, gat3_bl, gat3_wr, gat3_br, gat3_att, gat3_bias,
           gat4_wl, gat4_bl, gat4_wr, gat4_br, gat4_att, gat4_bias):
    f32 = jnp.float32
    p = (x[0, :, :NH * STRIDE, :NW * STRIDE]
         .reshape(IN_CH, NH, KS, NW, KS)
         .transpose(1, 3, 0, 2, 4)
         .reshape(N, IN_CH * KS * KS))
    wc_t = conv1_w.reshape(REPR, IN_CH * KS * KS).T
    emb = _embed(p, wc_t, conv1_b[None, :], fc2_w.T, fc2_b[None, :])

    nbr8, mskF = _topk(emb)
    idxF = jnp.zeros((3, PADN), jnp.int32).at[:, :N].set(nbr8[:, :K].T)

    mskS = jnp.asarray(_SP_MSK8)
    selS = jnp.asarray(_SP_SEL)

    ex8 = jnp.repeat(jnp.eye(HEADS, dtype=f32), REPR, axis=1)

    hS = _spatial(emb, mskS, selS, gat3_wl.T, gat3_wr.T,
                  _block_diag_att(gat3_att), ex8, gat3_bias[None, :],
                  gat4_wl.T, gat4_bl[None, :], gat4_wr.T, gat4_br[None, :],
                  gat4_att, gat4_bias[None, :])

    gF = _sc_gather(emb, idxF, REPR)
    xl2, xr2 = _gat8f(emb, gF, mskF, gat1_wl.T, gat1_wr.T,
                      _block_diag_att(gat1_att), ex8, gat1_bias[None, :],
                      gat2_wl.T, gat2_bl[None, :], gat2_wr.T,
                      gat2_br[None, :])
    g2 = _sc_gather(xl2, idxF, REPR)

    return _gat1f_fc3(xl2, xr2, g2, mskF, gat2_att, gat2_bias[None, :], hS,
                      fc3_w[:, :REPR].T, fc3_w[:, REPR:].T, fc3_b[None, :])
